# baseline (speedup 1.0000x reference)
import functools

import jax
import jax.numpy as jnp
from jax import lax
from jax.experimental import pallas as pl
from jax.experimental.pallas import tpu as pltpu
from jax.experimental.pallas import tpu_sc as plsc

B = 4096
L = 200
D = 32
NUM_WORKERS = 32
BBLOCK = B // NUM_WORKERS
NBUF = 8
NGROUPS = L // NBUF


def _sc_body(idx_hbm, table_hbm, out_hbm, idx_v, bufs, tb0, tb1, tb2, tb3,
             gs0, gs1, gs2, gs3, gs4, gs5, gs6, gs7, ss0, ss1, ss2, ss3):
    gsems = (gs0, gs1, gs2, gs3, gs4, gs5, gs6, gs7)
    tbufs = (tb0, tb1, tb2, tb3)
    ssems = (ss0, ss1, ss2, ss3)
    wid = lax.axis_index("s") * 2 + lax.axis_index("c")
    b0 = pl.multiple_of(wid * BBLOCK, BBLOCK)

    pltpu.sync_copy(idx_hbm.at[:, pl.ds(b0, BBLOCK)], idx_v)

    def fire(l, k):
        pltpu.async_copy(table_hbm.at[idx_v.at[l]], bufs.at[k], gsems[k])

    def wait_gather(k):
        pltpu.make_async_copy(table_hbm.at[pl.ds(0, BBLOCK)], bufs.at[k],
                              gsems[k]).wait()

    def store(l, p):
        pltpu.async_copy(tbufs[p], out_hbm.at[l, :, pl.ds(b0, BBLOCK)],
                         ssems[p])

    def wait_store(p):
        pltpu.make_async_copy(tbufs[p], out_hbm.at[0, :, pl.ds(b0, BBLOCK)],
                              ssems[p]).wait()

    lanes = lax.iota(jnp.int32, 16)
    zmask = (lanes > 0).astype(jnp.float32)
    lanes16 = [lanes + 16 * v for v in range(BBLOCK // 16)]

    def transpose_block(k, p):
        buf = bufs.at[k]
        for d in range(D):
            dvec = lanes * 0 + d
            for v in range(BBLOCK // 16):
                val = plsc.load_gather(buf, [lanes16[v], dvec])
                tbufs[p][d, pl.ds(16 * v, 16)] = val

        @pl.when(wid == 0)
        def _():
            for d in range(D):
                tbufs[p][d, pl.ds(0, 16)] = tbufs[p][d, pl.ds(0, 16)] * zmask

    for k in range(NBUF):
        fire(k, k)

    def group(g, carry):
        l0 = pl.multiple_of(g * NBUF, NBUF)
        for k in range(NBUF):
            l = l0 + k
            p = k % 4
            wait_gather(k)

            @pl.when(g + (1 if k >= 4 else 0) > 0)
            def _():
                wait_store(p)
            transpose_block(k, p)

            @pl.when(l + NBUF < L)
            def _():
                fire(l + NBUF, k)
            store(l, p)
        return carry

    lax.fori_loop(0, NGROUPS, group, 0)
    for p in range(4):
        wait_store(p)


@jax.jit
def _sc_gather(idx_t, table):
    mesh = plsc.VectorSubcoreMesh(core_axis_name="c", subcore_axis_name="s")
    run = functools.partial(
        pl.kernel,
        mesh=mesh,
        out_type=jax.ShapeDtypeStruct((L, D, B), jnp.float32),
        scratch_types=[
            pltpu.VMEM((L, BBLOCK), jnp.int32),
            pltpu.VMEM((NBUF, BBLOCK, D), jnp.float32),
            pltpu.VMEM((D, BBLOCK), jnp.float32),
            pltpu.VMEM((D, BBLOCK), jnp.float32),
            pltpu.VMEM((D, BBLOCK), jnp.float32),
            pltpu.VMEM((D, BBLOCK), jnp.float32),
            pltpu.SemaphoreType.DMA,
            pltpu.SemaphoreType.DMA,
            pltpu.SemaphoreType.DMA,
            pltpu.SemaphoreType.DMA,
            pltpu.SemaphoreType.DMA,
            pltpu.SemaphoreType.DMA,
            pltpu.SemaphoreType.DMA,
            pltpu.SemaphoreType.DMA,
            pltpu.SemaphoreType.DMA,
            pltpu.SemaphoreType.DMA,
            pltpu.SemaphoreType.DMA,
            pltpu.SemaphoreType.DMA,
        ],
        compiler_params=pltpu.CompilerParams(use_tc_tiling_on_sc=False,
                                             needs_layout_passes=False),
    )(_sc_body)
    return run(idx_t, table)


def kernel(y, table):
    yt = y.astype(jnp.int32).T
    idx_t = jnp.concatenate(
        [jnp.zeros((L, 1), jnp.int32), yt[:, :-1]], axis=1)
    out = _sc_gather(idx_t, table)
    return jnp.transpose(out, (2, 0, 1))


# speedup vs baseline: 1.2899x; 1.2899x over previous
import functools

import jax
import jax.numpy as jnp
from jax import lax
from jax.experimental import pallas as pl
from jax.experimental.pallas import tpu as pltpu
from jax.experimental.pallas import tpu_sc as plsc

B = 4096
L = 200
D = 32
N = B * L
NUM_WORKERS = 32
ROWS_PER_WORKER = N // NUM_WORKERS
GATHER_ROWS = 128
K = 10
CHUNK = GATHER_ROWS * K
NUM_CHUNKS = ROWS_PER_WORKER // CHUNK
NUM_STEPS = NUM_CHUNKS // 2


def _sc_body(idx_hbm, table_hbm, out_hbm, idx_v, rows0, rows1, gsem0, gsem1,
             ssem0, ssem1):
    wid = lax.axis_index("s") * 2 + lax.axis_index("c")
    base = pl.multiple_of(wid * ROWS_PER_WORKER, CHUNK)
    idx_row0 = pl.multiple_of(base // GATHER_ROWS, 8)

    pltpu.sync_copy(idx_hbm.at[pl.ds(idx_row0, ROWS_PER_WORKER // GATHER_ROWS)],
                    idx_v)

    def fire(c, buf, gsem):
        for j in range(K):
            pltpu.async_copy(
                table_hbm.at[idx_v.at[c * K + j]],
                buf.at[pl.ds(j * GATHER_ROWS, GATHER_ROWS)],
                gsem,
            )

    def drain_gathers(buf, gsem):
        pltpu.make_async_copy(table_hbm.at[pl.ds(0, CHUNK)], buf, gsem).wait()

    def store(c, buf, ssem):
        row0 = pl.multiple_of(base + c * CHUNK, CHUNK)
        pltpu.async_copy(buf, out_hbm.at[pl.ds(row0, CHUNK)], ssem)

    def wait_store(buf, ssem):
        pltpu.make_async_copy(buf, out_hbm.at[pl.ds(base, CHUNK)], ssem).wait()

    fire(0, rows0, gsem0)

    def step(i, carry):
        c0 = pl.multiple_of(2 * i, 2)

        @pl.when(i > 0)
        def _():
            wait_store(rows1, ssem1)
        fire(c0 + 1, rows1, gsem1)
        drain_gathers(rows0, gsem0)
        store(c0, rows0, ssem0)

        @pl.when(i < NUM_STEPS - 1)
        def _():
            wait_store(rows0, ssem0)
            fire(c0 + 2, rows0, gsem0)
        drain_gathers(rows1, gsem1)
        store(c0 + 1, rows1, ssem1)
        return carry

    lax.fori_loop(0, NUM_STEPS, step, 0)
    wait_store(rows0, ssem0)
    wait_store(rows1, ssem1)

    @pl.when(wid == 0)
    def _():
        zero = jnp.zeros((16,), jnp.float32)

        def zrow(i, carry):
            rows0[i, pl.ds(0, 16)] = zero
            rows0[i, pl.ds(16, 16)] = zero
            return carry

        lax.fori_loop(0, L, zrow, 0)
        pltpu.sync_copy(rows0.at[pl.ds(0, L)], out_hbm.at[pl.ds(0, L)])


@jax.jit
def _sc_gather(idx_rows, table):
    mesh = plsc.VectorSubcoreMesh(core_axis_name="c", subcore_axis_name="s")
    run = functools.partial(
        pl.kernel,
        mesh=mesh,
        out_type=jax.ShapeDtypeStruct((N, D), jnp.float32),
        scratch_types=[
            pltpu.VMEM((ROWS_PER_WORKER // GATHER_ROWS, GATHER_ROWS), jnp.int32),
            pltpu.VMEM((CHUNK, D), jnp.float32),
            pltpu.VMEM((CHUNK, D), jnp.float32),
            pltpu.SemaphoreType.DMA,
            pltpu.SemaphoreType.DMA,
            pltpu.SemaphoreType.DMA,
            pltpu.SemaphoreType.DMA,
        ],
        compiler_params=pltpu.CompilerParams(use_tc_tiling_on_sc=False),
    )(_sc_body)
    return run(idx_rows, table)


def kernel(y, table):
    yf = y.reshape(-1).astype(jnp.int32)
    idx = jnp.concatenate([jnp.zeros((L,), jnp.int32), yf[:-L]])
    idx_rows = idx.reshape(N // GATHER_ROWS, GATHER_ROWS)
    out = _sc_gather(idx_rows, table)
    return out.reshape(B, L, D)
